# Initial kernel scaffold; baseline (speedup 1.0000x reference)
import jax
import jax.numpy as jnp
from jax.experimental import pallas as pl


def kernel(bbox_input, label_input, bbox_target, label_target):
    raise NotImplementedError("write your pallas kernel here")


# speedup vs baseline: 3.6515x; 3.6515x over previous
import functools

import numpy as np

import jax
import jax.numpy as jnp
from jax import lax
from jax.experimental import pallas as pl
from jax.experimental.pallas import tpu as pltpu

NEG_RATIO = 3
SIGN = np.uint32(0x80000000)


def _keys_from_masked(masked):
    b = lax.bitcast_convert_type(masked, jnp.uint32)
    return jnp.where(b >= SIGN, ~b, b | SIGN)


def _vals_from_keys(u):
    b = jnp.where(u >= SIGN, u ^ SIGN, ~u)
    return lax.bitcast_convert_type(b, jnp.float32)


def _stage1_body(bb_in_ref, bb_tg_ref, li_ref, lt_ref, keys_ref, acc_ref):
    b = pl.program_id(0)
    C, A = li_ref.shape[1], li_ref.shape[2]

    lt = lt_ref[0]
    posf = (lt > 0).astype(jnp.float32)
    npos = jnp.sum(posf)

    d = bb_in_ref[0] - bb_tg_ref[0]
    ad = jnp.abs(d)
    sl1 = jnp.where(ad < 1.0, 0.5 * d * d, ad - 0.5)
    sl1_pos = jnp.sum(sl1 * posf)

    li = li_ref[0]
    cid = lax.broadcasted_iota(jnp.int32, (C, A), 0)
    onehot = (cid == lt).astype(jnp.float32)
    label_loss = -jnp.sum(li * onehot, axis=0, keepdims=True)
    pos_ll = jnp.sum(label_loss * posf)

    masked = label_loss * (posf - 1.0)
    keys_ref[...] = _keys_from_masked(masked)[:, None, :]

    lane = lax.broadcasted_iota(jnp.int32, (1, 128), 1)
    contrib = jnp.where(lane == 0, sl1_pos,
                        jnp.where(lane == 1, pos_ll,
                                  jnp.where(lane == 2, npos, 0.0)))

    @pl.when(b == 0)
    def _():
        acc_ref[...] = jnp.zeros_like(acc_ref)

    acc_ref[...] += contrib


def _stage2_body(keys_ref, lt_ref, out_ref):
    A = keys_ref.shape[2]
    u = keys_ref[:, 0, :]
    npos = jnp.sum((lt_ref[:, 0, :] > 0).astype(jnp.int32), axis=1,
                   keepdims=True)
    kv = jnp.minimum(NEG_RATIO * npos, A)

    def step(i, p):
        mid = p | (jnp.uint32(1) << (jnp.uint32(31) - i.astype(jnp.uint32)))
        cnt = jnp.sum((u < mid).astype(jnp.int32), axis=1, keepdims=True)
        return jnp.where(cnt >= kv, p, mid)

    p = lax.fori_loop(0, 32, step, jnp.zeros_like(kv, dtype=jnp.uint32))

    ltm = u < p
    c_lt = jnp.sum(ltm.astype(jnp.int32), axis=1, keepdims=True)
    masked = _vals_from_keys(u)
    sum_lt = jnp.sum(jnp.where(ltm, masked, 0.0), axis=1, keepdims=True)
    thr = _vals_from_keys(p)
    row_sel = sum_lt + (kv - c_lt).astype(jnp.float32) * thr
    row_sel = jnp.where(kv > 0, row_sel, 0.0)

    lane = lax.broadcasted_iota(jnp.int32, (1, 128), 1)
    out_ref[...] = jnp.where(lane == 0, jnp.sum(row_sel), 0.0)


@jax.jit
def kernel(bbox_input, label_input, bbox_target, label_target):
    B, A, _ = bbox_input.shape
    C = label_input.shape[1]
    lt = label_target.astype(jnp.int32).reshape(B, 1, A)
    bb_in = jnp.transpose(bbox_input, (0, 2, 1))
    bb_tg = jnp.transpose(bbox_target, (0, 2, 1))

    keys, acc = pl.pallas_call(
        _stage1_body,
        grid=(B,),
        in_specs=[
            pl.BlockSpec((1, 4, A), lambda b: (b, 0, 0)),
            pl.BlockSpec((1, 4, A), lambda b: (b, 0, 0)),
            pl.BlockSpec((1, C, A), lambda b: (b, 0, 0)),
            pl.BlockSpec((1, 1, A), lambda b: (b, 0, 0)),
        ],
        out_specs=[
            pl.BlockSpec((1, 1, A), lambda b: (b, 0, 0)),
            pl.BlockSpec((1, 128), lambda b: (0, 0)),
        ],
        out_shape=[
            jax.ShapeDtypeStruct((B, 1, A), jnp.uint32),
            jax.ShapeDtypeStruct((1, 128), jnp.float32),
        ],
        compiler_params=pltpu.CompilerParams(
            dimension_semantics=("arbitrary",),
        ),
    )(bb_in, bb_tg, label_input, lt)

    sel = pl.pallas_call(
        _stage2_body,
        in_specs=[
            pl.BlockSpec((B, 1, A), lambda: (0, 0, 0)),
            pl.BlockSpec((B, 1, A), lambda: (0, 0, 0)),
        ],
        out_specs=pl.BlockSpec((1, 128), lambda: (0, 0)),
        out_shape=jax.ShapeDtypeStruct((1, 128), jnp.float32),
    )(keys, lt)

    sl1_pos, pos_ll, npos = acc[0, 0], acc[0, 1], acc[0, 2]
    return (sl1_pos + pos_ll - sel[0, 0]) / npos
